# Initial kernel scaffold; baseline (speedup 1.0000x reference)
import jax
import jax.numpy as jnp
from jax.experimental import pallas as pl


def kernel(x, W_down, W_up, W_depth, W_router, ops, W_read, W_wkey, W_wgate, W_wval, slots):
    raise NotImplementedError("write your pallas kernel here")


# speedup vs baseline: 3.9585x; 3.9585x over previous
import math
import functools

import jax
import jax.numpy as jnp
from jax.experimental import pallas as pl

D_MODEL = 1024
D_REASON = 128
N_OPS = 32
TOP_K = 2
N_SLOTS = 16
DEPTHS = (1, 3, 8)
NEG_INF = -1e30


def _quant_prep_kernel(ops_ref, wr_ref, wd_ref, ops_out, wr_out, wd_out):
    ops = ops_ref[...]
    s = jnp.maximum(jnp.mean(jnp.abs(ops), axis=1, keepdims=True), 1e-5)
    ops_out[...] = jnp.clip(jnp.round(ops / s), -1.0, 1.0) * s

    wr = wr_ref[...]
    sr = jnp.maximum(jnp.mean(jnp.abs(wr)), 1e-5)
    wr_out[...] = jnp.clip(jnp.round(wr / sr), -1.0, 1.0) * sr

    wd = wd_ref[...]
    sd = jnp.maximum(jnp.mean(jnp.abs(wd)), 1e-5)
    wd_out[...] = jnp.clip(jnp.round(wd / sd), -1.0, 1.0) * sd


def _dot_nt(a, b):
    return jax.lax.dot_general(
        a, b, (((1,), (1,)), ((), ())), preferred_element_type=jnp.float32)


def _dot_nn(a, b):
    return jax.lax.dot_general(
        a, b, (((1,), (0,)), ((), ())), preferred_element_type=jnp.float32)


def _main_kernel(x_ref, wdown_ref, wup_ref, wd_eff_ref, wr_eff_ref,
                 ops_flat_ref, wread_ref, wwkey_ref, wwgate_ref, wwval_ref,
                 slots_ref, out_ref):
    x = x_ref[...]
    blk = x.shape[0]
    inv_sqrt_d = 1.0 / math.sqrt(D_REASON)

    reason = _dot_nt(x, wdown_ref[...])

    wd_eff = wd_eff_ref[...]
    dl0 = jnp.sum(reason * wd_eff[0:1, :], axis=1, keepdims=True)
    dl1 = jnp.sum(reason * wd_eff[1:2, :], axis=1, keepdims=True)
    dl2 = jnp.sum(reason * wd_eff[2:3, :], axis=1, keepdims=True)
    dm = jnp.maximum(dl0, jnp.maximum(dl1, dl2))
    e0 = jnp.exp(dl0 - dm)
    e1 = jnp.exp(dl1 - dm)
    e2 = jnp.exp(dl2 - dm)
    dz = e0 + e1 + e2
    p0, p1, p2 = e0 / dz, e1 / dz, e2 / dz

    wr_eff = wr_eff_ref[...]
    ops_flat = ops_flat_ref[...]
    wread = wread_ref[...]
    wwkey = wwkey_ref[...]
    wwgate = wwgate_ref[...]
    wwval = wwval_ref[...]
    slots = slots_ref[...]

    iota_ops = jax.lax.broadcasted_iota(jnp.int32, (blk, N_OPS), 1)
    iota_full = jax.lax.broadcasted_iota(
        jnp.int32, (blk, N_OPS * D_REASON), 1) // D_REASON

    state = reason
    results = []

    for step in range(DEPTHS[-1]):
        logits = _dot_nt(state, wr_eff)
        m0 = jnp.max(logits, axis=1, keepdims=True)
        i0 = jnp.min(jnp.where(logits == m0, iota_ops, N_OPS),
                     axis=1, keepdims=True)
        masked = jnp.where(iota_ops == i0, NEG_INF, logits)
        m1 = jnp.max(masked, axis=1, keepdims=True)
        i1 = jnp.min(jnp.where(masked == m1, iota_ops, N_OPS),
                     axis=1, keepdims=True)
        e = jnp.exp(m1 - m0)
        w0 = 1.0 / (1.0 + e)
        w1 = 1.0 - w0

        t_all = _dot_nt(state, ops_flat)
        coef = (1e-5 / N_OPS) \
            + jnp.where(iota_full == i0, w0, 0.0) \
            + jnp.where(iota_full == i1, w1, 0.0)
        r = t_all * coef
        width = N_OPS * D_REASON
        while width > D_REASON:
            half = width // 2
            r = r[:, :half] + r[:, half:width]
            width = half
        op_out = r

        key = _dot_nt(state, wread)
        scores = _dot_nt(key, slots) * inv_sqrt_d
        sm = jnp.max(scores, axis=1, keepdims=True)
        se = jnp.exp(scores - sm)
        attn = se / jnp.sum(se, axis=1, keepdims=True)
        mem = _dot_nn(attn, slots)

        wk = _dot_nt(state, wwkey)
        wg = jax.nn.sigmoid(jnp.sum(state * wwgate, axis=1, keepdims=True))
        wv = _dot_nt(state, wwval)
        wscores = _dot_nt(wk, slots) * inv_sqrt_d
        wsm = jnp.max(wscores, axis=1, keepdims=True)
        wse = jnp.exp(wscores - wsm)
        aw = wse / jnp.sum(wse, axis=1, keepdims=True)
        wsig = wg * wv + 0.1 * _dot_nn(aw, slots)

        state = state + op_out + mem + 0.1 * wsig
        if (step + 1) in DEPTHS:
            results.append(state)

    blended = p0 * results[0] + p1 * results[1] + p2 * results[2]
    out_ref[...] = x + _dot_nt(blended, wup_ref[...])


def kernel(x, W_down, W_up, W_depth, W_router, ops, W_read, W_wkey, W_wgate,
           W_wval, slots):
    B, S, _ = x.shape
    T = B * S

    ops_eff, wr_eff, wd_eff = pl.pallas_call(
        _quant_prep_kernel,
        out_shape=(
            jax.ShapeDtypeStruct((N_OPS, D_REASON * D_REASON), jnp.float32),
            jax.ShapeDtypeStruct((N_OPS, D_REASON), jnp.float32),
            jax.ShapeDtypeStruct((len(DEPTHS), D_REASON), jnp.float32),
        ),
    )(ops.reshape(N_OPS, D_REASON * D_REASON), W_router, W_depth)

    ops_flat = ops_eff.reshape(N_OPS * D_REASON, D_REASON)

    x2 = x.reshape(T, D_MODEL)
    BLK = 512
    grid = (T // BLK,)

    full = lambda shape: pl.BlockSpec(shape, lambda i: (0, 0))
    out = pl.pallas_call(
        _main_kernel,
        grid=grid,
        in_specs=[
            pl.BlockSpec((BLK, D_MODEL), lambda i: (i, 0)),
            full((D_REASON, D_MODEL)),
            full((D_MODEL, D_REASON)),
            full((len(DEPTHS), D_REASON)),
            full((N_OPS, D_REASON)),
            full((N_OPS * D_REASON, D_REASON)),
            full((D_REASON, D_REASON)),
            full((D_REASON, D_REASON)),
            full((1, D_REASON)),
            full((D_REASON, D_REASON)),
            full((N_SLOTS, D_REASON)),
        ],
        out_specs=pl.BlockSpec((BLK, D_MODEL), lambda i: (i, 0)),
        out_shape=jax.ShapeDtypeStruct((T, D_MODEL), jnp.float32),
    )(x2, W_down, W_up, wd_eff, wr_eff, ops_flat, W_read, W_wkey, W_wgate,
      W_wval, slots)

    return out.reshape(B, S, D_MODEL)
